# baseline (speedup 1.0000x reference)
import functools

import jax
import jax.numpy as jnp
from jax import lax
from jax.experimental import pallas as pl
from jax.experimental.pallas import tpu as pltpu
from jax.experimental.pallas import tpu_sc as plsc

_NC = 2
_NS = 16
_NW = _NC * _NS


def _sc_copy(in_hbm, out_hbm):
    n_rows = in_hbm.shape[0]
    rows_per_w = (n_rows // _NW) // 8 * 8
    last_rows = n_rows - (_NW - 1) * rows_per_w
    wid = lax.axis_index("s") * _NC + lax.axis_index("c")
    base = pl.multiple_of(wid * rows_per_w, 8)

    @pl.when(wid < _NW - 1)
    def _():
        pltpu.sync_copy(
            in_hbm.at[pl.ds(base, rows_per_w)],
            out_hbm.at[pl.ds(base, rows_per_w)],
        )

    @pl.when(wid == _NW - 1)
    def _():
        pltpu.sync_copy(
            in_hbm.at[pl.ds(base, last_rows)],
            out_hbm.at[pl.ds(base, last_rows)],
        )


def kernel(embeddings):
    mesh = plsc.VectorSubcoreMesh(core_axis_name="c", subcore_axis_name="s")
    run = pl.kernel(
        _sc_copy,
        out_type=jax.ShapeDtypeStruct(embeddings.shape, embeddings.dtype),
        mesh=mesh,
    )
    return run(embeddings)


# speedup vs baseline: 17.1025x; 17.1025x over previous
import jax
import jax.numpy as jnp
from jax import lax
from jax.experimental import pallas as pl
from jax.experimental.pallas import tpu as pltpu
from jax.experimental.pallas import tpu_sc as plsc

_NC = 2
_NS = 16
_NW = _NC * _NS

_ROWS = 1000000
_DIM = 32
_RPW = (_ROWS // _NW) // 8 * 8
_TAIL_BASE = _NW * _RPW
_TAIL_ROWS = _ROWS - _TAIL_BASE
_CHUNK = 1008
_N_CHUNKS = _RPW // _CHUNK


def _sc_copy(in_hbm, out_hbm, buf):
    wid = lax.axis_index("s") * _NC + lax.axis_index("c")
    base = pl.multiple_of(wid * _RPW, 8)
    for k in range(_N_CHUNKS):
        src = in_hbm.at[pl.ds(base + k * _CHUNK, _CHUNK)]
        dst = out_hbm.at[pl.ds(base + k * _CHUNK, _CHUNK)]
        pltpu.sync_copy(src, buf)
        pltpu.sync_copy(buf, dst)

    @pl.when(wid == 0)
    def _():
        tail = buf.at[pl.ds(0, _TAIL_ROWS)]
        pltpu.sync_copy(in_hbm.at[pl.ds(_TAIL_BASE, _TAIL_ROWS)], tail)
        pltpu.sync_copy(tail, out_hbm.at[pl.ds(_TAIL_BASE, _TAIL_ROWS)])


def kernel(embeddings):
    mesh = plsc.VectorSubcoreMesh(core_axis_name="c", subcore_axis_name="s")
    run = pl.kernel(
        _sc_copy,
        out_type=jax.ShapeDtypeStruct(embeddings.shape, embeddings.dtype),
        mesh=mesh,
        scratch_types=[pltpu.VMEM((_CHUNK, _DIM), jnp.float32)],
    )
    return run(embeddings)
